# baseline (speedup 1.0000x reference)
import functools

import jax
import jax.numpy as jnp
from jax import lax
from jax.experimental import pallas as pl
from jax.experimental.pallas import tpu as pltpu
from jax.experimental.pallas import tpu_sc as plsc

N_NODES = 10000
N_EDGES = 320000
D = 128

NC = 2
NS = 16
NW = NC * NS
CHUNK = 96
NCH = -(-(N_EDGES // NW) // CHUNK)
E_PAD = NW * NCH * CHUNK
N_PAD = 10240
ROWS_PER_TILE = N_PAD // NS


def _seg_sum_body(x_hbm, src_hbm, dst_hbm, zeros_hbm, out_hbm,
                  acc, sidx0, sidx1, sidx2, didx0, didx1, didx2,
                  rows0, rows1, rows2,
                  semi0, semi1, semi2, semg0, semg1, semg2):
    cid = lax.axis_index("c")
    sid = lax.axis_index("s")
    wid = sid * NC + cid
    base = wid * NCH * CHUNK
    sidx = (sidx0, sidx1, sidx2)
    didx = (didx0, didx1, didx2)
    rows = (rows0, rows1, rows2)
    semi = (semi0, semi1, semi2)
    semg = (semg0, semg1, semg2)

    def load_idx(g, b):
        pltpu.async_copy(src_hbm.at[pl.ds(base + g * CHUNK, CHUNK)],
                         sidx[b], semi[b])
        pltpu.async_copy(dst_hbm.at[pl.ds(base + g * CHUNK, CHUNK)],
                         didx[b], semi[b])

    def wait_idx(b):
        pltpu.make_async_copy(src_hbm.at[pl.ds(base, CHUNK)], sidx[b],
                              semi[b]).wait()
        pltpu.make_async_copy(dst_hbm.at[pl.ds(base, CHUNK)], didx[b],
                              semi[b]).wait()

    def gather(b):
        pltpu.async_copy(x_hbm.at[sidx[b]], rows[b], semg[b])

    def wait_gather_scatter(b):
        pltpu.make_async_copy(x_hbm.at[sidx[b]], rows[b], semg[b]).wait()
        pltpu.sync_copy(rows[b], acc.at[didx[b]], add=True)

    load_idx(0, 0)
    load_idx(1, 1)
    wait_idx(0)
    gather(0)
    wait_idx(1)
    gather(1)
    load_idx(2, 2)
    pltpu.sync_copy(zeros_hbm, acc.at[pl.ds(sid * ROWS_PER_TILE, ROWS_PER_TILE)])
    plsc.subcore_barrier()

    def step(h, carry):
        g0 = 3 * h
        for j in range(3):
            gj = g0 + j
            bj = j
            bn = (j + 2) % 3

            @pl.when(gj + 2 < NCH)
            def _():
                wait_idx(bn)
                gather(bn)

            wait_gather_scatter(bj)

            @pl.when(gj + 3 < NCH)
            def _():
                load_idx(gj + 3, bj)

        return carry

    lax.fori_loop(0, NCH // 3, step, 0)
    for b in range(NCH % 3):
        wait_gather_scatter(b)
    plsc.subcore_barrier()

    r0 = sid * ROWS_PER_TILE
    pltpu.sync_copy(acc.at[pl.ds(r0, ROWS_PER_TILE)],
                    out_hbm.at[pl.ds(cid * N_PAD + r0, ROWS_PER_TILE)])


_seg_sum = pl.kernel(
    _seg_sum_body,
    out_type=jax.ShapeDtypeStruct((NC * N_PAD, D), jnp.float32),
    mesh=plsc.VectorSubcoreMesh(core_axis_name="c", subcore_axis_name="s"),
    scratch_types=[
        pltpu.VMEM_SHARED((N_PAD, D), jnp.float32),
        pltpu.VMEM((CHUNK,), jnp.int32),
        pltpu.VMEM((CHUNK,), jnp.int32),
        pltpu.VMEM((CHUNK,), jnp.int32),
        pltpu.VMEM((CHUNK,), jnp.int32),
        pltpu.VMEM((CHUNK,), jnp.int32),
        pltpu.VMEM((CHUNK,), jnp.int32),
        pltpu.VMEM((CHUNK, D), jnp.float32),
        pltpu.VMEM((CHUNK, D), jnp.float32),
        pltpu.VMEM((CHUNK, D), jnp.float32),
        pltpu.SemaphoreType.DMA,
        pltpu.SemaphoreType.DMA,
        pltpu.SemaphoreType.DMA,
        pltpu.SemaphoreType.DMA,
        pltpu.SemaphoreType.DMA,
        pltpu.SemaphoreType.DMA,
    ],
)


ROWS_BLK = 1000
GRID = N_NODES // ROWS_BLK


def _combine_mid_body(p0_ref, p1_ref, x_ref, wrel_ref, wroot_ref, b_ref, o_ref):
    agg = p0_ref[0] + p1_ref[0]
    y = (jnp.dot(agg, wrel_ref[...], preferred_element_type=jnp.float32)
         + jnp.dot(x_ref[...], wroot_ref[...], preferred_element_type=jnp.float32)
         + b_ref[...])
    o_ref[...] = jnp.where(y > 0, y, 0.01 * y)


def _combine_last_body(p0_ref, p1_ref, x_ref, wrel_ref, wroot_ref, b_ref,
                       skip_ref, o_ref):
    agg = p0_ref[0] + p1_ref[0]
    y = (jnp.dot(agg, wrel_ref[...], preferred_element_type=jnp.float32)
         + jnp.dot(x_ref[...], wroot_ref[...], preferred_element_type=jnp.float32)
         + b_ref[...])
    y = jnp.where(y > 0, y, 0.01 * y) + skip_ref[...]
    part = jnp.sum(y, axis=0, keepdims=True)

    @pl.when(pl.program_id(0) == 0)
    def _():
        o_ref[...] = jnp.zeros_like(o_ref)

    o_ref[...] += part


def _decoder_body(s_ref, wdec_ref, wlin_ref, o_ref):
    mean = s_ref[...] * (1.0 / N_NODES)
    d = jnp.dot(mean, wdec_ref[...], preferred_element_type=jnp.float32)
    d = jnp.where(d > 0, d, 0.001 * d)
    logits = jnp.dot(d, wlin_ref[...], preferred_element_type=jnp.float32)
    m = jnp.max(logits, axis=-1, keepdims=True)
    e = jnp.exp(logits - m)
    o_ref[...] = e / jnp.sum(e, axis=-1, keepdims=True)


def _row_spec():
    return pl.BlockSpec((ROWS_BLK, D), lambda i: (i, 0))


def _p_spec(c):
    return pl.BlockSpec((1, ROWS_BLK, D), lambda i: (c, i, 0))


_W_SPEC = pl.BlockSpec((D, D), lambda i: (0, 0))
_B_SPEC = pl.BlockSpec((1, D), lambda i: (0, 0))

_combine_mid = pl.pallas_call(
    _combine_mid_body,
    grid=(GRID,),
    in_specs=[_p_spec(0), _p_spec(1),
              _row_spec(), _W_SPEC, _W_SPEC, _B_SPEC],
    out_specs=_row_spec(),
    out_shape=jax.ShapeDtypeStruct((N_NODES, D), jnp.float32),
)

_combine_last = pl.pallas_call(
    _combine_last_body,
    grid=(GRID,),
    in_specs=[_p_spec(0), _p_spec(1),
              _row_spec(), _W_SPEC, _W_SPEC, _B_SPEC, _row_spec()],
    out_specs=pl.BlockSpec((1, D), lambda i: (0, 0)),
    out_shape=jax.ShapeDtypeStruct((1, D), jnp.float32),
)

_decoder = pl.pallas_call(
    _decoder_body,
    in_specs=[pl.BlockSpec((1, D), lambda: (0, 0)),
              pl.BlockSpec((D, 64), lambda: (0, 0)),
              pl.BlockSpec((64, 16), lambda: (0, 0))],
    out_specs=pl.BlockSpec((1, 16), lambda: (0, 0)),
    out_shape=jax.ShapeDtypeStruct((1, 16), jnp.float32),
)


def kernel(x, edge_index, batch, W_rel_0, b_rel_0, W_root_0, W_rel_1, b_rel_1,
           W_root_1, W_rel_2, b_rel_2, W_root_2, W_rel_3, b_rel_3, W_root_3,
           W_dec_0, W_lin):
    pad = E_PAD - N_EDGES
    src = edge_index[0]
    dst = edge_index[1]
    if pad:
        src = jnp.concatenate([src, jnp.zeros((pad,), jnp.int32)])
        pad_dst = N_NODES + (jnp.arange(pad, dtype=jnp.int32)
                             % (N_PAD - N_NODES))
        dst = jnp.concatenate([dst, pad_dst])
    zeros = jnp.zeros((ROWS_PER_TILE, D), jnp.float32)
    W_rels = (W_rel_0, W_rel_1, W_rel_2, W_rel_3)
    b_rels = (b_rel_0.reshape(1, D), b_rel_1.reshape(1, D),
              b_rel_2.reshape(1, D), b_rel_3.reshape(1, D))
    W_roots = (W_root_0, W_root_1, W_root_2, W_root_3)

    outs = []
    for i in range(3):
        p = _seg_sum(x, src, dst, zeros).reshape(NC, N_PAD, D)
        x = _combine_mid(p, p, x, W_rels[i], W_roots[i], b_rels[i])
        outs.append(x)
    p = _seg_sum(x, src, dst, zeros).reshape(NC, N_PAD, D)
    sums = _combine_last(p, p, x, W_rels[3], W_roots[3], b_rels[3], outs[1])
    out = _decoder(sums, W_dec_0, W_lin)
    return out.reshape(16)


# speedup vs baseline: 1.1220x; 1.1220x over previous
import functools

import jax
import jax.numpy as jnp
from jax import lax
from jax.experimental import pallas as pl
from jax.experimental.pallas import tpu as pltpu
from jax.experimental.pallas import tpu_sc as plsc

N_NODES = 10000
N_EDGES = 320000
D = 128

NC = 2
NS = 16
NW = NC * NS
CHUNK = 64
NCH = -(-(N_EDGES // NW) // CHUNK)
E_PAD = NW * NCH * CHUNK
N_PAD = 10240
ROWS_PER_TILE = N_PAD // NS


def _seg_sum_body(x_hbm, src_hbm, dst_hbm, zeros_hbm, out_hbm,
                  acc, sidx0, sidx1, sidx2, didx0, didx1, didx2,
                  rows0, rows1, rows2,
                  semi0, semi1, semi2, semg0, semg1, semg2):
    cid = lax.axis_index("c")
    sid = lax.axis_index("s")
    wid = sid * NC + cid
    base = wid * NCH * CHUNK
    sidx = (sidx0, sidx1, sidx2)
    didx = (didx0, didx1, didx2)
    rows = (rows0, rows1, rows2)
    semi = (semi0, semi1, semi2)
    semg = (semg0, semg1, semg2)

    def load_idx(g, b):
        pltpu.async_copy(src_hbm.at[pl.ds(base + g * CHUNK, CHUNK)],
                         sidx[b], semi[b])
        pltpu.async_copy(dst_hbm.at[pl.ds(base + g * CHUNK, CHUNK)],
                         didx[b], semi[b])

    def wait_idx(b):
        pltpu.make_async_copy(src_hbm.at[pl.ds(base, CHUNK)], sidx[b],
                              semi[b]).wait()
        pltpu.make_async_copy(dst_hbm.at[pl.ds(base, CHUNK)], didx[b],
                              semi[b]).wait()

    def gather(b):
        pltpu.async_copy(x_hbm.at[sidx[b]], rows[b], semg[b])

    def wait_gather_scatter(b):
        pltpu.make_async_copy(x_hbm.at[sidx[b]], rows[b], semg[b]).wait()
        pltpu.sync_copy(rows[b], acc.at[didx[b]], add=True)

    load_idx(0, 0)
    load_idx(1, 1)
    wait_idx(0)
    gather(0)
    wait_idx(1)
    gather(1)
    load_idx(2, 2)
    pltpu.sync_copy(zeros_hbm, acc.at[pl.ds(sid * ROWS_PER_TILE, ROWS_PER_TILE)])
    plsc.subcore_barrier()

    def step(h, carry):
        g0 = 3 * h
        for j in range(3):
            gj = g0 + j
            bj = j
            bn = (j + 2) % 3

            @pl.when(gj + 2 < NCH)
            def _():
                wait_idx(bn)
                gather(bn)

            wait_gather_scatter(bj)

            @pl.when(gj + 3 < NCH)
            def _():
                load_idx(gj + 3, bj)

        return carry

    lax.fori_loop(0, NCH // 3, step, 0)
    for b in range(NCH % 3):
        wait_gather_scatter(b)
    plsc.subcore_barrier()

    r0 = sid * ROWS_PER_TILE
    pltpu.sync_copy(acc.at[pl.ds(r0, ROWS_PER_TILE)],
                    out_hbm.at[pl.ds(cid * N_PAD + r0, ROWS_PER_TILE)])


_seg_sum = pl.kernel(
    _seg_sum_body,
    out_type=jax.ShapeDtypeStruct((NC * N_PAD, D), jnp.float32),
    mesh=plsc.VectorSubcoreMesh(core_axis_name="c", subcore_axis_name="s"),
    scratch_types=[
        pltpu.VMEM_SHARED((N_PAD, D), jnp.float32),
        pltpu.VMEM((CHUNK,), jnp.int32),
        pltpu.VMEM((CHUNK,), jnp.int32),
        pltpu.VMEM((CHUNK,), jnp.int32),
        pltpu.VMEM((CHUNK,), jnp.int32),
        pltpu.VMEM((CHUNK,), jnp.int32),
        pltpu.VMEM((CHUNK,), jnp.int32),
        pltpu.VMEM((CHUNK, D), jnp.float32),
        pltpu.VMEM((CHUNK, D), jnp.float32),
        pltpu.VMEM((CHUNK, D), jnp.float32),
        pltpu.SemaphoreType.DMA,
        pltpu.SemaphoreType.DMA,
        pltpu.SemaphoreType.DMA,
        pltpu.SemaphoreType.DMA,
        pltpu.SemaphoreType.DMA,
        pltpu.SemaphoreType.DMA,
    ],
)


ROWS_BLK = 1000
GRID = N_NODES // ROWS_BLK


def _combine_mid_body(p0_ref, p1_ref, x_ref, wrel_ref, wroot_ref, b_ref, o_ref):
    agg = p0_ref[0] + p1_ref[0]
    y = (jnp.dot(agg, wrel_ref[...], preferred_element_type=jnp.float32)
         + jnp.dot(x_ref[...], wroot_ref[...], preferred_element_type=jnp.float32)
         + b_ref[...])
    o_ref[...] = jnp.where(y > 0, y, 0.01 * y)


def _combine_last_body(p0_ref, p1_ref, x_ref, wrel_ref, wroot_ref, b_ref,
                       skip_ref, o_ref):
    agg = p0_ref[0] + p1_ref[0]
    y = (jnp.dot(agg, wrel_ref[...], preferred_element_type=jnp.float32)
         + jnp.dot(x_ref[...], wroot_ref[...], preferred_element_type=jnp.float32)
         + b_ref[...])
    y = jnp.where(y > 0, y, 0.01 * y) + skip_ref[...]
    part = jnp.sum(y, axis=0, keepdims=True)

    @pl.when(pl.program_id(0) == 0)
    def _():
        o_ref[...] = jnp.zeros_like(o_ref)

    o_ref[...] += part


def _decoder_body(s_ref, wdec_ref, wlin_ref, o_ref):
    mean = s_ref[...] * (1.0 / N_NODES)
    d = jnp.dot(mean, wdec_ref[...], preferred_element_type=jnp.float32)
    d = jnp.where(d > 0, d, 0.001 * d)
    logits = jnp.dot(d, wlin_ref[...], preferred_element_type=jnp.float32)
    m = jnp.max(logits, axis=-1, keepdims=True)
    e = jnp.exp(logits - m)
    o_ref[...] = e / jnp.sum(e, axis=-1, keepdims=True)


def _row_spec():
    return pl.BlockSpec((ROWS_BLK, D), lambda i: (i, 0))


def _p_spec(c):
    return pl.BlockSpec((1, ROWS_BLK, D), lambda i: (c, i, 0))


_W_SPEC = pl.BlockSpec((D, D), lambda i: (0, 0))
_B_SPEC = pl.BlockSpec((1, D), lambda i: (0, 0))

_combine_mid = pl.pallas_call(
    _combine_mid_body,
    grid=(GRID,),
    in_specs=[_p_spec(0), _p_spec(1),
              _row_spec(), _W_SPEC, _W_SPEC, _B_SPEC],
    out_specs=_row_spec(),
    out_shape=jax.ShapeDtypeStruct((N_NODES, D), jnp.float32),
)

_combine_last = pl.pallas_call(
    _combine_last_body,
    grid=(GRID,),
    in_specs=[_p_spec(0), _p_spec(1),
              _row_spec(), _W_SPEC, _W_SPEC, _B_SPEC, _row_spec()],
    out_specs=pl.BlockSpec((1, D), lambda i: (0, 0)),
    out_shape=jax.ShapeDtypeStruct((1, D), jnp.float32),
)

_decoder = pl.pallas_call(
    _decoder_body,
    in_specs=[pl.BlockSpec((1, D), lambda: (0, 0)),
              pl.BlockSpec((D, 64), lambda: (0, 0)),
              pl.BlockSpec((64, 16), lambda: (0, 0))],
    out_specs=pl.BlockSpec((1, 16), lambda: (0, 0)),
    out_shape=jax.ShapeDtypeStruct((1, 16), jnp.float32),
)


def kernel(x, edge_index, batch, W_rel_0, b_rel_0, W_root_0, W_rel_1, b_rel_1,
           W_root_1, W_rel_2, b_rel_2, W_root_2, W_rel_3, b_rel_3, W_root_3,
           W_dec_0, W_lin):
    pad = E_PAD - N_EDGES
    src = edge_index[0]
    dst = edge_index[1]
    if pad:
        src = jnp.concatenate([src, jnp.zeros((pad,), jnp.int32)])
        pad_dst = N_NODES + (jnp.arange(pad, dtype=jnp.int32)
                             % (N_PAD - N_NODES))
        dst = jnp.concatenate([dst, pad_dst])
    zeros = jnp.zeros((ROWS_PER_TILE, D), jnp.float32)
    W_rels = (W_rel_0, W_rel_1, W_rel_2, W_rel_3)
    b_rels = (b_rel_0.reshape(1, D), b_rel_1.reshape(1, D),
              b_rel_2.reshape(1, D), b_rel_3.reshape(1, D))
    W_roots = (W_root_0, W_root_1, W_root_2, W_root_3)

    outs = []
    for i in range(3):
        p = _seg_sum(x, src, dst, zeros).reshape(NC, N_PAD, D)
        x = _combine_mid(p, p, x, W_rels[i], W_roots[i], b_rels[i])
        outs.append(x)
    p = _seg_sum(x, src, dst, zeros).reshape(NC, N_PAD, D)
    sums = _combine_last(p, p, x, W_rels[3], W_roots[3], b_rels[3], outs[1])
    out = _decoder(sums, W_dec_0, W_lin)
    return out.reshape(16)


# speedup vs baseline: 1.6504x; 1.4709x over previous
import functools

import jax
import jax.numpy as jnp
from jax import lax
from jax.experimental import pallas as pl
from jax.experimental.pallas import tpu as pltpu
from jax.experimental.pallas import tpu_sc as plsc

N_NODES = 10000
N_EDGES = 320000
D = 128

NC = 2
NS = 16
NW = NC * NS
CHUNK = 80
NCH = -(-(N_EDGES // NW) // CHUNK)
E_PAD = NW * NCH * CHUNK
N_PAD = 10240
ROWS_PER_TILE = N_PAD // NS


def _seg_sum_body(x_hbm, src_hbm, dst_hbm, zeros_hbm, out_hbm,
                  acc, sidx0, sidx1, sidx2, didx0, didx1, didx2,
                  rows0, rows1, rows2,
                  semi0, semi1, semi2, semg0, semg1, semg2):
    cid = lax.axis_index("c")
    sid = lax.axis_index("s")
    wid = sid * NC + cid
    base = wid * NCH * CHUNK
    sidx = (sidx0, sidx1, sidx2)
    didx = (didx0, didx1, didx2)
    rows = (rows0, rows1, rows2)
    semi = (semi0, semi1, semi2)
    semg = (semg0, semg1, semg2)

    def load_idx(g, b):
        pltpu.async_copy(src_hbm.at[pl.ds(base + g * CHUNK, CHUNK)],
                         sidx[b], semi[b])
        pltpu.async_copy(dst_hbm.at[pl.ds(base + g * CHUNK, CHUNK)],
                         didx[b], semi[b])

    def wait_idx(b):
        pltpu.make_async_copy(src_hbm.at[pl.ds(base, CHUNK)], sidx[b],
                              semi[b]).wait()
        pltpu.make_async_copy(dst_hbm.at[pl.ds(base, CHUNK)], didx[b],
                              semi[b]).wait()

    def gather(b):
        pltpu.async_copy(x_hbm.at[sidx[b]], rows[b], semg[b])

    def wait_gather_scatter(b):
        pltpu.make_async_copy(x_hbm.at[sidx[b]], rows[b], semg[b]).wait()
        pltpu.sync_copy(rows[b], acc.at[didx[b]], add=True)

    load_idx(0, 0)
    load_idx(1, 1)
    wait_idx(0)
    gather(0)
    wait_idx(1)
    gather(1)
    load_idx(2, 2)
    pltpu.sync_copy(zeros_hbm, acc.at[pl.ds(sid * ROWS_PER_TILE, ROWS_PER_TILE)])
    plsc.subcore_barrier()

    def step(h, carry):
        g0 = 3 * h
        for j in range(3):
            gj = g0 + j
            bj = j
            bn = (j + 2) % 3

            @pl.when(gj + 2 < NCH)
            def _():
                wait_idx(bn)
                gather(bn)

            wait_gather_scatter(bj)

            @pl.when(gj + 3 < NCH)
            def _():
                load_idx(gj + 3, bj)

        return carry

    lax.fori_loop(0, NCH // 3, step, 0)
    for b in range(NCH % 3):
        wait_gather_scatter(b)
    plsc.subcore_barrier()

    r0 = sid * ROWS_PER_TILE
    pltpu.sync_copy(acc.at[pl.ds(r0, ROWS_PER_TILE)],
                    out_hbm.at[pl.ds(cid * N_PAD + r0, ROWS_PER_TILE)])


_seg_sum = pl.kernel(
    _seg_sum_body,
    out_type=jax.ShapeDtypeStruct((NC * N_PAD, D), jnp.float32),
    mesh=plsc.VectorSubcoreMesh(core_axis_name="c", subcore_axis_name="s"),
    scratch_types=[
        pltpu.VMEM_SHARED((N_PAD, D), jnp.float32),
        pltpu.VMEM((CHUNK,), jnp.int32),
        pltpu.VMEM((CHUNK,), jnp.int32),
        pltpu.VMEM((CHUNK,), jnp.int32),
        pltpu.VMEM((CHUNK,), jnp.int32),
        pltpu.VMEM((CHUNK,), jnp.int32),
        pltpu.VMEM((CHUNK,), jnp.int32),
        pltpu.VMEM((CHUNK, D), jnp.float32),
        pltpu.VMEM((CHUNK, D), jnp.float32),
        pltpu.VMEM((CHUNK, D), jnp.float32),
        pltpu.SemaphoreType.DMA,
        pltpu.SemaphoreType.DMA,
        pltpu.SemaphoreType.DMA,
        pltpu.SemaphoreType.DMA,
        pltpu.SemaphoreType.DMA,
        pltpu.SemaphoreType.DMA,
    ],
)


ROWS_BLK = 1000
GRID = N_NODES // ROWS_BLK


def _combine_mid_body(p0_ref, p1_ref, x_ref, wrel_ref, wroot_ref, b_ref, o_ref):
    agg = p0_ref[0] + p1_ref[0]
    y = (jnp.dot(agg, wrel_ref[...], preferred_element_type=jnp.float32)
         + jnp.dot(x_ref[...], wroot_ref[...], preferred_element_type=jnp.float32)
         + b_ref[...])
    o_ref[...] = jnp.where(y > 0, y, 0.01 * y)


def _combine_last_body(p0_ref, p1_ref, x_ref, wrel_ref, wroot_ref, b_ref,
                       skip_ref, o_ref):
    agg = p0_ref[0] + p1_ref[0]
    y = (jnp.dot(agg, wrel_ref[...], preferred_element_type=jnp.float32)
         + jnp.dot(x_ref[...], wroot_ref[...], preferred_element_type=jnp.float32)
         + b_ref[...])
    y = jnp.where(y > 0, y, 0.01 * y) + skip_ref[...]
    part = jnp.sum(y, axis=0, keepdims=True)

    @pl.when(pl.program_id(0) == 0)
    def _():
        o_ref[...] = jnp.zeros_like(o_ref)

    o_ref[...] += part


def _decoder_body(s_ref, wdec_ref, wlin_ref, o_ref):
    mean = s_ref[...] * (1.0 / N_NODES)
    d = jnp.dot(mean, wdec_ref[...], preferred_element_type=jnp.float32)
    d = jnp.where(d > 0, d, 0.001 * d)
    logits = jnp.dot(d, wlin_ref[...], preferred_element_type=jnp.float32)
    m = jnp.max(logits, axis=-1, keepdims=True)
    e = jnp.exp(logits - m)
    o_ref[...] = e / jnp.sum(e, axis=-1, keepdims=True)


def _row_spec():
    return pl.BlockSpec((ROWS_BLK, D), lambda i: (i, 0))


def _p_spec(c):
    return pl.BlockSpec((1, ROWS_BLK, D), lambda i: (c, i, 0))


_W_SPEC = pl.BlockSpec((D, D), lambda i: (0, 0))
_B_SPEC = pl.BlockSpec((1, D), lambda i: (0, 0))

_combine_mid = pl.pallas_call(
    _combine_mid_body,
    grid=(GRID,),
    in_specs=[_p_spec(0), _p_spec(1),
              _row_spec(), _W_SPEC, _W_SPEC, _B_SPEC],
    out_specs=_row_spec(),
    out_shape=jax.ShapeDtypeStruct((N_NODES, D), jnp.float32),
)

_combine_last = pl.pallas_call(
    _combine_last_body,
    grid=(GRID,),
    in_specs=[_p_spec(0), _p_spec(1),
              _row_spec(), _W_SPEC, _W_SPEC, _B_SPEC, _row_spec()],
    out_specs=pl.BlockSpec((1, D), lambda i: (0, 0)),
    out_shape=jax.ShapeDtypeStruct((1, D), jnp.float32),
)

_decoder = pl.pallas_call(
    _decoder_body,
    in_specs=[pl.BlockSpec((1, D), lambda: (0, 0)),
              pl.BlockSpec((D, 64), lambda: (0, 0)),
              pl.BlockSpec((64, 16), lambda: (0, 0))],
    out_specs=pl.BlockSpec((1, 16), lambda: (0, 0)),
    out_shape=jax.ShapeDtypeStruct((1, 16), jnp.float32),
)


def kernel(x, edge_index, batch, W_rel_0, b_rel_0, W_root_0, W_rel_1, b_rel_1,
           W_root_1, W_rel_2, b_rel_2, W_root_2, W_rel_3, b_rel_3, W_root_3,
           W_dec_0, W_lin):
    pad = E_PAD - N_EDGES
    src = edge_index[0]
    dst = edge_index[1]
    if pad:
        src = jnp.concatenate([src, jnp.zeros((pad,), jnp.int32)])
        pad_dst = N_NODES + (jnp.arange(pad, dtype=jnp.int32)
                             % (N_PAD - N_NODES))
        dst = jnp.concatenate([dst, pad_dst])
    zeros = jnp.zeros((ROWS_PER_TILE, D), jnp.float32)
    W_rels = (W_rel_0, W_rel_1, W_rel_2, W_rel_3)
    b_rels = (b_rel_0.reshape(1, D), b_rel_1.reshape(1, D),
              b_rel_2.reshape(1, D), b_rel_3.reshape(1, D))
    W_roots = (W_root_0, W_root_1, W_root_2, W_root_3)

    outs = []
    for i in range(3):
        p = _seg_sum(x, src, dst, zeros).reshape(NC, N_PAD, D)
        x = _combine_mid(p, p, x, W_rels[i], W_roots[i], b_rels[i])
        outs.append(x)
    p = _seg_sum(x, src, dst, zeros).reshape(NC, N_PAD, D)
    sums = _combine_last(p, p, x, W_rels[3], W_roots[3], b_rels[3], outs[1])
    out = _decoder(sums, W_dec_0, W_lin)
    return out.reshape(16)


# speedup vs baseline: 1.6531x; 1.0017x over previous
import functools

import jax
import jax.numpy as jnp
from jax import lax
from jax.experimental import pallas as pl
from jax.experimental.pallas import tpu as pltpu
from jax.experimental.pallas import tpu_sc as plsc

N_NODES = 10000
N_EDGES = 320000
D = 128

NC = 2
NS = 16
NW = NC * NS
CHUNK = 80
NCH = -(-(N_EDGES // NW) // CHUNK)
E_PAD = NW * NCH * CHUNK
N_PAD = 10240
ROWS_PER_TILE = N_PAD // NS


NBUF = 4


def _seg_sum_body(x_hbm, src_hbm, dst_hbm, zeros_hbm, out_hbm,
                  acc, *scr):
    cid = lax.axis_index("c")
    sid = lax.axis_index("s")
    wid = sid * NC + cid
    base = wid * NCH * CHUNK
    sidx = scr[0:NBUF]
    didx = scr[NBUF:2 * NBUF]
    rows = scr[2 * NBUF:3 * NBUF]
    semi = scr[3 * NBUF:4 * NBUF]
    semg = scr[4 * NBUF:5 * NBUF]

    def load_idx(g, b):
        pltpu.async_copy(src_hbm.at[pl.ds(base + g * CHUNK, CHUNK)],
                         sidx[b], semi[b])
        pltpu.async_copy(dst_hbm.at[pl.ds(base + g * CHUNK, CHUNK)],
                         didx[b], semi[b])

    def wait_idx(b):
        pltpu.make_async_copy(src_hbm.at[pl.ds(base, CHUNK)], sidx[b],
                              semi[b]).wait()
        pltpu.make_async_copy(dst_hbm.at[pl.ds(base, CHUNK)], didx[b],
                              semi[b]).wait()

    def gather(b):
        pltpu.async_copy(x_hbm.at[sidx[b]], rows[b], semg[b])

    def wait_gather_scatter(b):
        pltpu.make_async_copy(x_hbm.at[sidx[b]], rows[b], semg[b]).wait()
        pltpu.sync_copy(rows[b], acc.at[didx[b]], add=True)

    for b in range(NBUF - 1):
        load_idx(b, b)
    for b in range(NBUF - 1):
        wait_idx(b)
        gather(b)
    load_idx(NBUF - 1, NBUF - 1)
    pltpu.sync_copy(zeros_hbm, acc.at[pl.ds(sid * ROWS_PER_TILE, ROWS_PER_TILE)])
    plsc.subcore_barrier()

    def step(h, carry):
        g0 = NBUF * h
        for j in range(NBUF):
            gj = g0 + j
            bn = (j + NBUF - 1) % NBUF

            @pl.when(gj + NBUF - 1 < NCH)
            def _():
                wait_idx(bn)
                gather(bn)

            wait_gather_scatter(j)

            @pl.when(gj + NBUF < NCH)
            def _():
                load_idx(gj + NBUF, j)

        return carry

    lax.fori_loop(0, NCH // NBUF, step, 0)
    for b in range(NCH % NBUF):
        wait_gather_scatter(b)
    plsc.subcore_barrier()

    r0 = sid * ROWS_PER_TILE
    pltpu.sync_copy(acc.at[pl.ds(r0, ROWS_PER_TILE)],
                    out_hbm.at[pl.ds(cid * N_PAD + r0, ROWS_PER_TILE)])


_seg_sum = pl.kernel(
    _seg_sum_body,
    out_type=jax.ShapeDtypeStruct((NC * N_PAD, D), jnp.float32),
    mesh=plsc.VectorSubcoreMesh(core_axis_name="c", subcore_axis_name="s"),
    scratch_types=(
        [pltpu.VMEM_SHARED((N_PAD, D), jnp.float32)]
        + [pltpu.VMEM((CHUNK,), jnp.int32) for _ in range(2 * NBUF)]
        + [pltpu.VMEM((CHUNK, D), jnp.float32) for _ in range(NBUF)]
        + [pltpu.SemaphoreType.DMA for _ in range(2 * NBUF)]
    ),
)


ROWS_BLK = 1000
GRID = N_NODES // ROWS_BLK


def _combine_mid_body(p0_ref, p1_ref, x_ref, wrel_ref, wroot_ref, b_ref, o_ref):
    agg = p0_ref[0] + p1_ref[0]
    y = (jnp.dot(agg, wrel_ref[...], preferred_element_type=jnp.float32)
         + jnp.dot(x_ref[...], wroot_ref[...], preferred_element_type=jnp.float32)
         + b_ref[...])
    o_ref[...] = jnp.where(y > 0, y, 0.01 * y)


def _combine_last_body(p0_ref, p1_ref, x_ref, wrel_ref, wroot_ref, b_ref,
                       skip_ref, o_ref):
    agg = p0_ref[0] + p1_ref[0]
    y = (jnp.dot(agg, wrel_ref[...], preferred_element_type=jnp.float32)
         + jnp.dot(x_ref[...], wroot_ref[...], preferred_element_type=jnp.float32)
         + b_ref[...])
    y = jnp.where(y > 0, y, 0.01 * y) + skip_ref[...]
    part = jnp.sum(y, axis=0, keepdims=True)

    @pl.when(pl.program_id(0) == 0)
    def _():
        o_ref[...] = jnp.zeros_like(o_ref)

    o_ref[...] += part


def _decoder_body(s_ref, wdec_ref, wlin_ref, o_ref):
    mean = s_ref[...] * (1.0 / N_NODES)
    d = jnp.dot(mean, wdec_ref[...], preferred_element_type=jnp.float32)
    d = jnp.where(d > 0, d, 0.001 * d)
    logits = jnp.dot(d, wlin_ref[...], preferred_element_type=jnp.float32)
    m = jnp.max(logits, axis=-1, keepdims=True)
    e = jnp.exp(logits - m)
    o_ref[...] = e / jnp.sum(e, axis=-1, keepdims=True)


def _row_spec():
    return pl.BlockSpec((ROWS_BLK, D), lambda i: (i, 0))


def _p_spec(c):
    return pl.BlockSpec((1, ROWS_BLK, D), lambda i: (c, i, 0))


_W_SPEC = pl.BlockSpec((D, D), lambda i: (0, 0))
_B_SPEC = pl.BlockSpec((1, D), lambda i: (0, 0))

_combine_mid = pl.pallas_call(
    _combine_mid_body,
    grid=(GRID,),
    in_specs=[_p_spec(0), _p_spec(1),
              _row_spec(), _W_SPEC, _W_SPEC, _B_SPEC],
    out_specs=_row_spec(),
    out_shape=jax.ShapeDtypeStruct((N_NODES, D), jnp.float32),
)

_combine_last = pl.pallas_call(
    _combine_last_body,
    grid=(GRID,),
    in_specs=[_p_spec(0), _p_spec(1),
              _row_spec(), _W_SPEC, _W_SPEC, _B_SPEC, _row_spec()],
    out_specs=pl.BlockSpec((1, D), lambda i: (0, 0)),
    out_shape=jax.ShapeDtypeStruct((1, D), jnp.float32),
)

_decoder = pl.pallas_call(
    _decoder_body,
    in_specs=[pl.BlockSpec((1, D), lambda: (0, 0)),
              pl.BlockSpec((D, 64), lambda: (0, 0)),
              pl.BlockSpec((64, 16), lambda: (0, 0))],
    out_specs=pl.BlockSpec((1, 16), lambda: (0, 0)),
    out_shape=jax.ShapeDtypeStruct((1, 16), jnp.float32),
)


def kernel(x, edge_index, batch, W_rel_0, b_rel_0, W_root_0, W_rel_1, b_rel_1,
           W_root_1, W_rel_2, b_rel_2, W_root_2, W_rel_3, b_rel_3, W_root_3,
           W_dec_0, W_lin):
    pad = E_PAD - N_EDGES
    src = edge_index[0]
    dst = edge_index[1]
    if pad:
        src = jnp.concatenate([src, jnp.zeros((pad,), jnp.int32)])
        pad_dst = N_NODES + (jnp.arange(pad, dtype=jnp.int32)
                             % (N_PAD - N_NODES))
        dst = jnp.concatenate([dst, pad_dst])
    zeros = jnp.zeros((ROWS_PER_TILE, D), jnp.float32)
    W_rels = (W_rel_0, W_rel_1, W_rel_2, W_rel_3)
    b_rels = (b_rel_0.reshape(1, D), b_rel_1.reshape(1, D),
              b_rel_2.reshape(1, D), b_rel_3.reshape(1, D))
    W_roots = (W_root_0, W_root_1, W_root_2, W_root_3)

    outs = []
    for i in range(3):
        p = _seg_sum(x, src, dst, zeros).reshape(NC, N_PAD, D)
        x = _combine_mid(p, p, x, W_rels[i], W_roots[i], b_rels[i])
        outs.append(x)
    p = _seg_sum(x, src, dst, zeros).reshape(NC, N_PAD, D)
    sums = _combine_last(p, p, x, W_rels[3], W_roots[3], b_rels[3], outs[1])
    out = _decoder(sums, W_dec_0, W_lin)
    return out.reshape(16)


# speedup vs baseline: 2.2345x; 1.3517x over previous
import functools

import jax
import jax.numpy as jnp
from jax import lax
from jax.experimental import pallas as pl
from jax.experimental.pallas import tpu as pltpu
from jax.experimental.pallas import tpu_sc as plsc

N_NODES = 10000
N_EDGES = 320000
D = 128

NC = 2
NS = 16
NW = NC * NS
CHUNK = 80
NCH = -(-(N_EDGES // NW) // CHUNK)
E_PAD = NW * NCH * CHUNK
N_PAD = 10240
ROWS_PER_TILE = N_PAD // NS


NBUF = 4


def _seg_sum_body(x_hbm, src_hbm, dst_hbm, zeros_hbm, out_hbm,
                  acc, *scr):
    cid = lax.axis_index("c")
    sid = lax.axis_index("s")
    wid = sid * NC + cid
    base = wid * NCH * CHUNK
    sidx = scr[0:NBUF]
    didx = scr[NBUF:2 * NBUF]
    rows = scr[2 * NBUF:3 * NBUF]
    semi = scr[3 * NBUF:4 * NBUF]
    semg = scr[4 * NBUF:5 * NBUF]
    sems = scr[5 * NBUF:6 * NBUF]

    def load_idx(g, b):
        pltpu.async_copy(src_hbm.at[pl.ds(base + g * CHUNK, CHUNK)],
                         sidx[b], semi[b])
        pltpu.async_copy(dst_hbm.at[pl.ds(base + g * CHUNK, CHUNK)],
                         didx[b], semi[b])

    def wait_idx(b):
        pltpu.make_async_copy(src_hbm.at[pl.ds(base, CHUNK)], sidx[b],
                              semi[b]).wait()
        pltpu.make_async_copy(dst_hbm.at[pl.ds(base, CHUNK)], didx[b],
                              semi[b]).wait()

    def gather(b):
        pltpu.async_copy(x_hbm.at[sidx[b]], rows[b], semg[b])

    def wait_gather_scatter(b):
        pltpu.make_async_copy(x_hbm.at[sidx[b]], rows[b], semg[b]).wait()
        pltpu.async_copy(rows[b], acc.at[didx[b]], sems[b], add=True)

    def wait_scatter(b):
        pltpu.make_async_copy(rows[b], acc.at[didx[b]], sems[b]).wait()

    for b in range(NBUF - 1):
        load_idx(b, b)
    for b in range(NBUF - 1):
        wait_idx(b)
        gather(b)
    load_idx(NBUF - 1, NBUF - 1)
    pltpu.sync_copy(zeros_hbm, acc.at[pl.ds(sid * ROWS_PER_TILE, ROWS_PER_TILE)])
    plsc.subcore_barrier()

    def step(h, carry):
        g0 = NBUF * h
        for j in range(NBUF):
            gj = g0 + j
            bn = (j + NBUF - 1) % NBUF

            @pl.when(gj + NBUF - 1 < NCH)
            def _():
                wait_idx(bn)

                @pl.when(gj > 0)
                def _():
                    wait_scatter(bn)

                gather(bn)

            wait_gather_scatter(j)

            @pl.when(gj + NBUF < NCH)
            def _():
                load_idx(gj + NBUF, j)

        return carry

    lax.fori_loop(0, NCH // NBUF, step, 0)
    for b in range(NCH % NBUF):
        wait_gather_scatter(b)
    for b in range(NBUF):
        wait_scatter(b)
    plsc.subcore_barrier()

    r0 = sid * ROWS_PER_TILE
    pltpu.sync_copy(acc.at[pl.ds(r0, ROWS_PER_TILE)],
                    out_hbm.at[pl.ds(cid * N_PAD + r0, ROWS_PER_TILE)])


_seg_sum = pl.kernel(
    _seg_sum_body,
    out_type=jax.ShapeDtypeStruct((NC * N_PAD, D), jnp.float32),
    mesh=plsc.VectorSubcoreMesh(core_axis_name="c", subcore_axis_name="s"),
    scratch_types=(
        [pltpu.VMEM_SHARED((N_PAD, D), jnp.float32)]
        + [pltpu.VMEM((CHUNK,), jnp.int32) for _ in range(2 * NBUF)]
        + [pltpu.VMEM((CHUNK, D), jnp.float32) for _ in range(NBUF)]
        + [pltpu.SemaphoreType.DMA for _ in range(3 * NBUF)]
    ),
)


ROWS_BLK = 1000
GRID = N_NODES // ROWS_BLK


def _combine_mid_body(p0_ref, p1_ref, x_ref, wrel_ref, wroot_ref, b_ref, o_ref):
    agg = p0_ref[0] + p1_ref[0]
    y = (jnp.dot(agg, wrel_ref[...], preferred_element_type=jnp.float32)
         + jnp.dot(x_ref[...], wroot_ref[...], preferred_element_type=jnp.float32)
         + b_ref[...])
    o_ref[...] = jnp.where(y > 0, y, 0.01 * y)


def _combine_last_body(p0_ref, p1_ref, x_ref, wrel_ref, wroot_ref, b_ref,
                       skip_ref, o_ref):
    agg = p0_ref[0] + p1_ref[0]
    y = (jnp.dot(agg, wrel_ref[...], preferred_element_type=jnp.float32)
         + jnp.dot(x_ref[...], wroot_ref[...], preferred_element_type=jnp.float32)
         + b_ref[...])
    y = jnp.where(y > 0, y, 0.01 * y) + skip_ref[...]
    part = jnp.sum(y, axis=0, keepdims=True)

    @pl.when(pl.program_id(0) == 0)
    def _():
        o_ref[...] = jnp.zeros_like(o_ref)

    o_ref[...] += part


def _decoder_body(s_ref, wdec_ref, wlin_ref, o_ref):
    mean = s_ref[...] * (1.0 / N_NODES)
    d = jnp.dot(mean, wdec_ref[...], preferred_element_type=jnp.float32)
    d = jnp.where(d > 0, d, 0.001 * d)
    logits = jnp.dot(d, wlin_ref[...], preferred_element_type=jnp.float32)
    m = jnp.max(logits, axis=-1, keepdims=True)
    e = jnp.exp(logits - m)
    o_ref[...] = e / jnp.sum(e, axis=-1, keepdims=True)


def _row_spec():
    return pl.BlockSpec((ROWS_BLK, D), lambda i: (i, 0))


def _p_spec(c):
    return pl.BlockSpec((1, ROWS_BLK, D), lambda i: (c, i, 0))


_W_SPEC = pl.BlockSpec((D, D), lambda i: (0, 0))
_B_SPEC = pl.BlockSpec((1, D), lambda i: (0, 0))

_combine_mid = pl.pallas_call(
    _combine_mid_body,
    grid=(GRID,),
    in_specs=[_p_spec(0), _p_spec(1),
              _row_spec(), _W_SPEC, _W_SPEC, _B_SPEC],
    out_specs=_row_spec(),
    out_shape=jax.ShapeDtypeStruct((N_NODES, D), jnp.float32),
)

_combine_last = pl.pallas_call(
    _combine_last_body,
    grid=(GRID,),
    in_specs=[_p_spec(0), _p_spec(1),
              _row_spec(), _W_SPEC, _W_SPEC, _B_SPEC, _row_spec()],
    out_specs=pl.BlockSpec((1, D), lambda i: (0, 0)),
    out_shape=jax.ShapeDtypeStruct((1, D), jnp.float32),
)

_decoder = pl.pallas_call(
    _decoder_body,
    in_specs=[pl.BlockSpec((1, D), lambda: (0, 0)),
              pl.BlockSpec((D, 64), lambda: (0, 0)),
              pl.BlockSpec((64, 16), lambda: (0, 0))],
    out_specs=pl.BlockSpec((1, 16), lambda: (0, 0)),
    out_shape=jax.ShapeDtypeStruct((1, 16), jnp.float32),
)


def kernel(x, edge_index, batch, W_rel_0, b_rel_0, W_root_0, W_rel_1, b_rel_1,
           W_root_1, W_rel_2, b_rel_2, W_root_2, W_rel_3, b_rel_3, W_root_3,
           W_dec_0, W_lin):
    pad = E_PAD - N_EDGES
    src = edge_index[0]
    dst = edge_index[1]
    if pad:
        src = jnp.concatenate([src, jnp.zeros((pad,), jnp.int32)])
        pad_dst = N_NODES + (jnp.arange(pad, dtype=jnp.int32)
                             % (N_PAD - N_NODES))
        dst = jnp.concatenate([dst, pad_dst])
    zeros = jnp.zeros((ROWS_PER_TILE, D), jnp.float32)
    W_rels = (W_rel_0, W_rel_1, W_rel_2, W_rel_3)
    b_rels = (b_rel_0.reshape(1, D), b_rel_1.reshape(1, D),
              b_rel_2.reshape(1, D), b_rel_3.reshape(1, D))
    W_roots = (W_root_0, W_root_1, W_root_2, W_root_3)

    outs = []
    for i in range(3):
        p = _seg_sum(x, src, dst, zeros).reshape(NC, N_PAD, D)
        x = _combine_mid(p, p, x, W_rels[i], W_roots[i], b_rels[i])
        outs.append(x)
    p = _seg_sum(x, src, dst, zeros).reshape(NC, N_PAD, D)
    sums = _combine_last(p, p, x, W_rels[3], W_roots[3], b_rels[3], outs[1])
    out = _decoder(sums, W_dec_0, W_lin)
    return out.reshape(16)


# speedup vs baseline: 2.3724x; 1.0617x over previous
import functools

import jax
import jax.numpy as jnp
from jax import lax
from jax.experimental import pallas as pl
from jax.experimental.pallas import tpu as pltpu
from jax.experimental.pallas import tpu_sc as plsc

N_NODES = 10000
N_EDGES = 320000
D = 128

NC = 2
NS = 16
NW = NC * NS
CHUNK = 80
NCH = -(-(N_EDGES // NW) // CHUNK)
E_PAD = NW * NCH * CHUNK
N_PAD = 10240
ROWS_PER_TILE = N_PAD // NS


NBUF = 4


def _seg_sum_body(x_hbm, src_hbm, dst_hbm, out_hbm,
                  acc, *scr):
    cid = lax.axis_index("c")
    sid = lax.axis_index("s")
    wid = sid * NC + cid
    base = wid * NCH * CHUNK
    sidx = scr[0:NBUF]
    didx = scr[NBUF:2 * NBUF]
    rows = scr[2 * NBUF:3 * NBUF]
    semi = scr[3 * NBUF:4 * NBUF]
    semg = scr[4 * NBUF:5 * NBUF]
    sems = scr[5 * NBUF:6 * NBUF]

    def load_idx(g, b):
        pltpu.async_copy(src_hbm.at[pl.ds(base + g * CHUNK, CHUNK)],
                         sidx[b], semi[b])
        pltpu.async_copy(dst_hbm.at[pl.ds(base + g * CHUNK, CHUNK)],
                         didx[b], semi[b])

    def wait_idx(b):
        pltpu.make_async_copy(src_hbm.at[pl.ds(base, CHUNK)], sidx[b],
                              semi[b]).wait()
        pltpu.make_async_copy(dst_hbm.at[pl.ds(base, CHUNK)], didx[b],
                              semi[b]).wait()

    def gather(b):
        pltpu.async_copy(x_hbm.at[sidx[b]], rows[b], semg[b])

    def wait_gather_scatter(b):
        pltpu.make_async_copy(x_hbm.at[sidx[b]], rows[b], semg[b]).wait()
        pltpu.async_copy(rows[b], acc.at[didx[b]], sems[b], add=True)

    def wait_scatter(b):
        pltpu.make_async_copy(rows[b], acc.at[didx[b]], sems[b]).wait()

    for b in range(NBUF - 1):
        load_idx(b, b)
    zbuf = rows[NBUF - 1]

    def zstore(i, carry):
        for j in range(D // 16):
            zbuf[i, pl.ds(j * 16, 16)] = jnp.zeros((16,), jnp.float32)
        return carry

    lax.fori_loop(0, CHUNK, zstore, 0)
    for b in range(NBUF - 1):
        wait_idx(b)
        gather(b)
    load_idx(NBUF - 1, NBUF - 1)
    for k in range(ROWS_PER_TILE // CHUNK):
        pltpu.sync_copy(
            zbuf, acc.at[pl.ds(sid * ROWS_PER_TILE + k * CHUNK, CHUNK)])
    plsc.subcore_barrier()

    def step(h, carry):
        g0 = NBUF * h
        for j in range(NBUF):
            gj = g0 + j
            bn = (j + NBUF - 1) % NBUF

            @pl.when(gj + NBUF - 1 < NCH)
            def _():
                wait_idx(bn)

                @pl.when(gj > 0)
                def _():
                    wait_scatter(bn)

                gather(bn)

            wait_gather_scatter(j)

            @pl.when(gj + NBUF < NCH)
            def _():
                load_idx(gj + NBUF, j)

        return carry

    lax.fori_loop(0, NCH // NBUF, step, 0)
    for b in range(NCH % NBUF):
        wait_gather_scatter(b)
    for b in range(NBUF):
        wait_scatter(b)
    plsc.subcore_barrier()

    r0 = sid * ROWS_PER_TILE
    pltpu.sync_copy(acc.at[pl.ds(r0, ROWS_PER_TILE)],
                    out_hbm.at[pl.ds(cid * N_PAD + r0, ROWS_PER_TILE)])


_seg_sum = pl.kernel(
    _seg_sum_body,
    out_type=jax.ShapeDtypeStruct((NC * N_PAD, D), jnp.float32),
    mesh=plsc.VectorSubcoreMesh(core_axis_name="c", subcore_axis_name="s",
                                num_cores=NC, num_subcores=NS),
    scratch_types=(
        [pltpu.VMEM_SHARED((N_PAD, D), jnp.float32)]
        + [pltpu.VMEM((CHUNK,), jnp.int32) for _ in range(2 * NBUF)]
        + [pltpu.VMEM((CHUNK, D), jnp.float32) for _ in range(NBUF)]
        + [pltpu.SemaphoreType.DMA for _ in range(3 * NBUF)]
    ),
)


ROWS_BLK = 1000
GRID = N_NODES // ROWS_BLK


def _combine_mid_body(p0_ref, p1_ref, x_ref, wrel_ref, wroot_ref, b_ref, o_ref):
    agg = p0_ref[0] + p1_ref[0]
    y = (jnp.dot(agg, wrel_ref[...], preferred_element_type=jnp.float32)
         + jnp.dot(x_ref[...], wroot_ref[...], preferred_element_type=jnp.float32)
         + b_ref[...])
    o_ref[...] = jnp.where(y > 0, y, 0.01 * y)


def _combine_last_body(p0_ref, p1_ref, x_ref, wrel_ref, wroot_ref, b_ref,
                       skip_ref, o_ref):
    agg = p0_ref[0] + p1_ref[0]
    y = (jnp.dot(agg, wrel_ref[...], preferred_element_type=jnp.float32)
         + jnp.dot(x_ref[...], wroot_ref[...], preferred_element_type=jnp.float32)
         + b_ref[...])
    y = jnp.where(y > 0, y, 0.01 * y) + skip_ref[...]
    part = jnp.sum(y, axis=0, keepdims=True)

    @pl.when(pl.program_id(0) == 0)
    def _():
        o_ref[...] = jnp.zeros_like(o_ref)

    o_ref[...] += part


def _decoder_body(s_ref, wdec_ref, wlin_ref, o_ref):
    mean = s_ref[...] * (1.0 / N_NODES)
    d = jnp.dot(mean, wdec_ref[...], preferred_element_type=jnp.float32)
    d = jnp.where(d > 0, d, 0.001 * d)
    logits = jnp.dot(d, wlin_ref[...], preferred_element_type=jnp.float32)
    m = jnp.max(logits, axis=-1, keepdims=True)
    e = jnp.exp(logits - m)
    o_ref[...] = e / jnp.sum(e, axis=-1, keepdims=True)


def _row_spec():
    return pl.BlockSpec((ROWS_BLK, D), lambda i: (i, 0))


def _p_spec(c):
    return pl.BlockSpec((1, ROWS_BLK, D), lambda i: (c, i, 0))


_W_SPEC = pl.BlockSpec((D, D), lambda i: (0, 0))
_B_SPEC = pl.BlockSpec((1, D), lambda i: (0, 0))

_combine_mid = pl.pallas_call(
    _combine_mid_body,
    grid=(GRID,),
    in_specs=[_p_spec(0), _p_spec(1),
              _row_spec(), _W_SPEC, _W_SPEC, _B_SPEC],
    out_specs=_row_spec(),
    out_shape=jax.ShapeDtypeStruct((N_NODES, D), jnp.float32),
)

_combine_last = pl.pallas_call(
    _combine_last_body,
    grid=(GRID,),
    in_specs=[_p_spec(0), _p_spec(1),
              _row_spec(), _W_SPEC, _W_SPEC, _B_SPEC, _row_spec()],
    out_specs=pl.BlockSpec((1, D), lambda i: (0, 0)),
    out_shape=jax.ShapeDtypeStruct((1, D), jnp.float32),
)

_decoder = pl.pallas_call(
    _decoder_body,
    in_specs=[pl.BlockSpec((1, D), lambda: (0, 0)),
              pl.BlockSpec((D, 64), lambda: (0, 0)),
              pl.BlockSpec((64, 16), lambda: (0, 0))],
    out_specs=pl.BlockSpec((1, 16), lambda: (0, 0)),
    out_shape=jax.ShapeDtypeStruct((1, 16), jnp.float32),
)


def kernel(x, edge_index, batch, W_rel_0, b_rel_0, W_root_0, W_rel_1, b_rel_1,
           W_root_1, W_rel_2, b_rel_2, W_root_2, W_rel_3, b_rel_3, W_root_3,
           W_dec_0, W_lin):
    pad = E_PAD - N_EDGES
    src = edge_index[0]
    dst = edge_index[1]
    if pad:
        src = jnp.concatenate([src, jnp.zeros((pad,), jnp.int32)])
        pad_dst = N_NODES + (jnp.arange(pad, dtype=jnp.int32)
                             % (N_PAD - N_NODES))
        dst = jnp.concatenate([dst, pad_dst])
    W_rels = (W_rel_0, W_rel_1, W_rel_2, W_rel_3)
    b_rels = (b_rel_0.reshape(1, D), b_rel_1.reshape(1, D),
              b_rel_2.reshape(1, D), b_rel_3.reshape(1, D))
    W_roots = (W_root_0, W_root_1, W_root_2, W_root_3)

    outs = []
    for i in range(3):
        p = _seg_sum(x, src, dst).reshape(NC, N_PAD, D)
        x = _combine_mid(p, p, x, W_rels[i], W_roots[i], b_rels[i])
        outs.append(x)
    p = _seg_sum(x, src, dst).reshape(NC, N_PAD, D)
    sums = _combine_last(p, p, x, W_rels[3], W_roots[3], b_rels[3], outs[1])
    out = _decoder(sums, W_dec_0, W_lin)
    return out.reshape(16)


# speedup vs baseline: 2.4375x; 1.0274x over previous
import functools

import jax
import jax.numpy as jnp
from jax import lax
from jax.experimental import pallas as pl
from jax.experimental.pallas import tpu as pltpu
from jax.experimental.pallas import tpu_sc as plsc

N_NODES = 10000
N_EDGES = 320000
D = 128

NC = 2
NS = 16
NW = NC * NS
CHUNK = 80
NCH = -(-(N_EDGES // NW) // CHUNK)
E_PAD = NW * NCH * CHUNK
N_PAD = 10240
ROWS_PER_TILE = N_PAD // NS


NBUF = 4


def _seg_sum_body(x_hbm, src_hbm, dst_hbm, out_hbm,
                  acc, *scr):
    cid = lax.axis_index("c")
    sid = lax.axis_index("s")
    wid = sid * NC + cid
    base = wid * NCH * CHUNK
    sidx = scr[0:NBUF]
    didx = scr[NBUF:2 * NBUF]
    rows = scr[2 * NBUF:3 * NBUF]
    semi = scr[3 * NBUF:4 * NBUF]
    semg = scr[4 * NBUF:5 * NBUF]
    sems = scr[5 * NBUF:6 * NBUF]

    def load_idx(g, b):
        pltpu.async_copy(src_hbm.at[pl.ds(base + g * CHUNK, CHUNK)],
                         sidx[b], semi[b])
        pltpu.async_copy(dst_hbm.at[pl.ds(base + g * CHUNK, CHUNK)],
                         didx[b], semi[b])

    def wait_idx(b):
        pltpu.make_async_copy(src_hbm.at[pl.ds(base, CHUNK)], sidx[b],
                              semi[b]).wait()
        pltpu.make_async_copy(dst_hbm.at[pl.ds(base, CHUNK)], didx[b],
                              semi[b]).wait()

    def gather(b):
        pltpu.async_copy(x_hbm.at[sidx[b]], rows[b], semg[b])

    def wait_gather_scatter(b):
        pltpu.make_async_copy(x_hbm.at[sidx[b]], rows[b], semg[b]).wait()
        pltpu.async_copy(rows[b], acc.at[didx[b]], sems[b], add=True)

    def wait_scatter(b):
        pltpu.make_async_copy(rows[b], acc.at[didx[b]], sems[b]).wait()

    for b in range(NBUF - 1):
        load_idx(b, b)
    zbuf = rows[NBUF - 1]

    def zstore(i, carry):
        for j in range(D // 16):
            zbuf[i, pl.ds(j * 16, 16)] = jnp.zeros((16,), jnp.float32)
        return carry

    lax.fori_loop(0, CHUNK, zstore, 0)
    for b in range(NBUF - 1):
        wait_idx(b)
        gather(b)
    load_idx(NBUF - 1, NBUF - 1)
    for k in range(ROWS_PER_TILE // CHUNK):
        pltpu.sync_copy(
            zbuf, acc.at[pl.ds(sid * ROWS_PER_TILE + k * CHUNK, CHUNK)])
    plsc.subcore_barrier()

    def step(h, carry):
        g0 = NBUF * h
        for j in range(NBUF):
            gj = g0 + j
            bn = (j + NBUF - 1) % NBUF

            @pl.when(gj + NBUF - 1 < NCH)
            def _():
                wait_idx(bn)

                @pl.when(gj > 0)
                def _():
                    wait_scatter(bn)

                gather(bn)

            wait_gather_scatter(j)

            @pl.when(gj + NBUF < NCH)
            def _():
                load_idx(gj + NBUF, j)

        return carry

    lax.fori_loop(0, NCH // NBUF, step, 0)
    for b in range(NCH % NBUF):
        wait_gather_scatter(b)
    for b in range(NBUF):
        wait_scatter(b)
    plsc.subcore_barrier()

    r0 = sid * ROWS_PER_TILE
    pltpu.sync_copy(acc.at[pl.ds(r0, ROWS_PER_TILE)],
                    out_hbm.at[pl.ds(cid * N_PAD + r0, ROWS_PER_TILE)])


_seg_sum = pl.kernel(
    _seg_sum_body,
    out_type=jax.ShapeDtypeStruct((NC * N_PAD, D), jnp.float32),
    mesh=plsc.VectorSubcoreMesh(core_axis_name="c", subcore_axis_name="s",
                                num_cores=NC, num_subcores=NS),
    scratch_types=(
        [pltpu.VMEM_SHARED((N_PAD, D), jnp.float32)]
        + [pltpu.VMEM((CHUNK,), jnp.int32) for _ in range(2 * NBUF)]
        + [pltpu.VMEM((CHUNK, D), jnp.float32) for _ in range(NBUF)]
        + [pltpu.SemaphoreType.DMA for _ in range(3 * NBUF)]
    ),
)


ROWS_BLK = 2000
GRID = N_NODES // ROWS_BLK


def _combine_mid_body(p0_ref, p1_ref, x_ref, wrel_ref, wroot_ref, b_ref, o_ref):
    agg = p0_ref[0] + p1_ref[0]
    y = (jnp.dot(agg, wrel_ref[...], preferred_element_type=jnp.float32)
         + jnp.dot(x_ref[...], wroot_ref[...], preferred_element_type=jnp.float32)
         + b_ref[...])
    o_ref[...] = jnp.where(y > 0, y, 0.01 * y)


def _combine_last_body(p0_ref, p1_ref, x_ref, wrel_ref, wroot_ref, b_ref,
                       skip_ref, o_ref):
    agg = p0_ref[0] + p1_ref[0]
    y = (jnp.dot(agg, wrel_ref[...], preferred_element_type=jnp.float32)
         + jnp.dot(x_ref[...], wroot_ref[...], preferred_element_type=jnp.float32)
         + b_ref[...])
    y = jnp.where(y > 0, y, 0.01 * y) + skip_ref[...]
    part = jnp.sum(y, axis=0, keepdims=True)

    @pl.when(pl.program_id(0) == 0)
    def _():
        o_ref[...] = jnp.zeros_like(o_ref)

    o_ref[...] += part


def _decoder_body(s_ref, wdec_ref, wlin_ref, o_ref):
    mean = s_ref[...] * (1.0 / N_NODES)
    d = jnp.dot(mean, wdec_ref[...], preferred_element_type=jnp.float32)
    d = jnp.where(d > 0, d, 0.001 * d)
    logits = jnp.dot(d, wlin_ref[...], preferred_element_type=jnp.float32)
    m = jnp.max(logits, axis=-1, keepdims=True)
    e = jnp.exp(logits - m)
    o_ref[...] = e / jnp.sum(e, axis=-1, keepdims=True)


def _row_spec():
    return pl.BlockSpec((ROWS_BLK, D), lambda i: (i, 0))


def _p_spec(c):
    return pl.BlockSpec((1, ROWS_BLK, D), lambda i: (c, i, 0))


_W_SPEC = pl.BlockSpec((D, D), lambda i: (0, 0))
_B_SPEC = pl.BlockSpec((1, D), lambda i: (0, 0))

_combine_mid = pl.pallas_call(
    _combine_mid_body,
    grid=(GRID,),
    in_specs=[_p_spec(0), _p_spec(1),
              _row_spec(), _W_SPEC, _W_SPEC, _B_SPEC],
    out_specs=_row_spec(),
    out_shape=jax.ShapeDtypeStruct((N_NODES, D), jnp.float32),
)

_combine_last = pl.pallas_call(
    _combine_last_body,
    grid=(GRID,),
    in_specs=[_p_spec(0), _p_spec(1),
              _row_spec(), _W_SPEC, _W_SPEC, _B_SPEC, _row_spec()],
    out_specs=pl.BlockSpec((1, D), lambda i: (0, 0)),
    out_shape=jax.ShapeDtypeStruct((1, D), jnp.float32),
)

_decoder = pl.pallas_call(
    _decoder_body,
    in_specs=[pl.BlockSpec((1, D), lambda: (0, 0)),
              pl.BlockSpec((D, 64), lambda: (0, 0)),
              pl.BlockSpec((64, 16), lambda: (0, 0))],
    out_specs=pl.BlockSpec((1, 16), lambda: (0, 0)),
    out_shape=jax.ShapeDtypeStruct((1, 16), jnp.float32),
)


def kernel(x, edge_index, batch, W_rel_0, b_rel_0, W_root_0, W_rel_1, b_rel_1,
           W_root_1, W_rel_2, b_rel_2, W_root_2, W_rel_3, b_rel_3, W_root_3,
           W_dec_0, W_lin):
    pad = E_PAD - N_EDGES
    src = edge_index[0]
    dst = edge_index[1]
    if pad:
        src = jnp.concatenate([src, jnp.zeros((pad,), jnp.int32)])
        pad_dst = N_NODES + (jnp.arange(pad, dtype=jnp.int32)
                             % (N_PAD - N_NODES))
        dst = jnp.concatenate([dst, pad_dst])
    W_rels = (W_rel_0, W_rel_1, W_rel_2, W_rel_3)
    b_rels = (b_rel_0.reshape(1, D), b_rel_1.reshape(1, D),
              b_rel_2.reshape(1, D), b_rel_3.reshape(1, D))
    W_roots = (W_root_0, W_root_1, W_root_2, W_root_3)

    outs = []
    for i in range(3):
        p = _seg_sum(x, src, dst).reshape(NC, N_PAD, D)
        x = _combine_mid(p, p, x, W_rels[i], W_roots[i], b_rels[i])
        outs.append(x)
    p = _seg_sum(x, src, dst).reshape(NC, N_PAD, D)
    sums = _combine_last(p, p, x, W_rels[3], W_roots[3], b_rels[3], outs[1])
    out = _decoder(sums, W_dec_0, W_lin)
    return out.reshape(16)
